# Initial kernel scaffold; baseline (speedup 1.0000x reference)
import jax
import jax.numpy as jnp
from jax.experimental import pallas as pl


def kernel(x, edge_index, W1, b1, W2, b2):
    raise NotImplementedError("write your pallas kernel here")


# speedup vs baseline: 11.0106x; 11.0106x over previous
import functools

import jax
import jax.numpy as jnp
from jax import lax
from jax.experimental import pallas as pl
from jax.experimental.pallas import tpu as pltpu
from jax.experimental.pallas import tpu_sc as plsc

N = 10000
NP = 10240
E = 320000
IN_DIM = 128
HID = 128
OUT = 64
NC, NS = 2, 16
NW = NC * NS
EPW = 10240
EP = NW * EPW
CHUNK = 128
NCHUNK = EPW // CHUNK
RPS = NP // NS
ROWB = 512
GRID = NP // ROWB

_MESH = plsc.VectorSubcoreMesh(core_axis_name="c", subcore_axis_name="s")


def _sc_deg_body(dst_hbm, p_hbm, idx_v, ones_v, z_v, psh):
    c = lax.axis_index("c")
    s = lax.axis_index("s")
    wid = s * NC + c
    for i in range(CHUNK // 16):
        ones_v[pl.ds(16 * i, 16)] = jnp.ones((16,), jnp.float32)

    def zb(i, _):
        z_v[pl.ds(i * 16, 16)] = jnp.zeros((16,), jnp.float32)
        return 0

    lax.fori_loop(0, RPS // 16, zb, 0)
    pltpu.sync_copy(z_v, psh.at[pl.ds(s * RPS, RPS)])
    plsc.subcore_barrier()

    pltpu.sync_copy(dst_hbm.at[pl.ds(wid * NCHUNK, NCHUNK)], idx_v)

    def step(j, _):
        pltpu.sync_copy(ones_v, psh.at[idx_v.at[j]], add=True)
        return 0

    lax.fori_loop(0, NCHUNK, step, 0)
    plsc.subcore_barrier()
    pltpu.sync_copy(psh.at[pl.ds(s * RPS, RPS)],
                    p_hbm.at[c, pl.ds(s * RPS, RPS)])


_sc_deg = pl.kernel(
    _sc_deg_body,
    out_type=jax.ShapeDtypeStruct((NC, NP), jnp.float32),
    mesh=_MESH,
    scratch_types=[
        pltpu.VMEM((NCHUNK, CHUNK), jnp.int32),
        pltpu.VMEM((CHUNK,), jnp.float32),
        pltpu.VMEM((RPS,), jnp.float32),
        pltpu.VMEM_SHARED((NP,), jnp.float32),
    ],
)


def _sc_agg_body(dim, src_hbm, dst_hbm, h_hbm, agg_hbm,
                 isrc_v, idst_v, rows_v, acc_sh, sem):
    c = lax.axis_index("c")
    s = lax.axis_index("s")
    wid = s * NC + c
    nv = dim // 16

    def zrow(i, _):
        for k in range(nv):
            rows_v[i, pl.ds(16 * k, 16)] = jnp.zeros((16,), jnp.float32)
        return 0

    lax.fori_loop(0, CHUNK, zrow, 0)
    for t in range(RPS // CHUNK):
        pltpu.sync_copy(rows_v, acc_sh.at[pl.ds(s * RPS + t * CHUNK, CHUNK)])
    plsc.subcore_barrier()

    pltpu.sync_copy(src_hbm.at[pl.ds(wid * NCHUNK, NCHUNK)], isrc_v)
    pltpu.sync_copy(dst_hbm.at[pl.ds(wid * NCHUNK, NCHUNK)], idst_v)

    def step(j, _):
        pltpu.async_copy(h_hbm.at[isrc_v.at[j]], rows_v, sem).wait()
        pltpu.sync_copy(rows_v, acc_sh.at[idst_v.at[j]], add=True)
        return 0

    lax.fori_loop(0, NCHUNK, step, 0)
    plsc.subcore_barrier()
    pltpu.sync_copy(acc_sh.at[pl.ds(s * RPS, RPS)],
                    agg_hbm.at[c, pl.ds(s * RPS, RPS)])


def _make_sc_agg(dim):
    return pl.kernel(
        functools.partial(_sc_agg_body, dim),
        out_type=jax.ShapeDtypeStruct((NC, NP, dim), jnp.float32),
        mesh=_MESH,
        scratch_types=[
            pltpu.VMEM((NCHUNK, CHUNK), jnp.int32),
            pltpu.VMEM((NCHUNK, CHUNK), jnp.int32),
            pltpu.VMEM((CHUNK, dim), jnp.float32),
            pltpu.VMEM_SHARED((NP, dim), jnp.float32),
            pltpu.SemaphoreType.DMA,
        ],
        compiler_params=pltpu.CompilerParams(use_tc_tiling_on_sc=False),
    )


_sc_agg_hid = _make_sc_agg(HID)
_sc_agg_out = _make_sc_agg(OUT)


def _dinv_body(p_ref, o_ref):
    o_ref[...] = lax.rsqrt(p_ref[0] + p_ref[1] + 1.0)


_tc_dinv = pl.pallas_call(
    _dinv_body,
    out_shape=jax.ShapeDtypeStruct((NP // 128, 128), jnp.float32),
)


def _mm1_body(x_ref, w_ref, dv_ref, o_ref):
    o_ref[...] = jnp.dot(x_ref[...], w_ref[...],
                         preferred_element_type=jnp.float32) * dv_ref[...]


_tc_mm1 = pl.pallas_call(
    _mm1_body,
    grid=(GRID,),
    in_specs=[
        pl.BlockSpec((ROWB, IN_DIM), lambda i: (i, 0)),
        pl.BlockSpec((IN_DIM, HID), lambda i: (0, 0)),
        pl.BlockSpec((ROWB, 1), lambda i: (i, 0)),
    ],
    out_specs=pl.BlockSpec((ROWB, HID), lambda i: (i, 0)),
    out_shape=jax.ShapeDtypeStruct((NP, HID), jnp.float32),
)


def _mid_body(agg_ref, h_ref, dv_ref, b_ref, w_ref, o_ref):
    t = (agg_ref[0] + agg_ref[1] + h_ref[...]) * dv_ref[...] + b_ref[...]
    t = jnp.maximum(t, 0.0)
    o_ref[...] = jnp.dot(t, w_ref[...],
                         preferred_element_type=jnp.float32) * dv_ref[...]


_tc_mid = pl.pallas_call(
    _mid_body,
    grid=(GRID,),
    in_specs=[
        pl.BlockSpec((NC, ROWB, HID), lambda i: (0, i, 0)),
        pl.BlockSpec((ROWB, HID), lambda i: (i, 0)),
        pl.BlockSpec((ROWB, 1), lambda i: (i, 0)),
        pl.BlockSpec((1, HID), lambda i: (0, 0)),
        pl.BlockSpec((HID, OUT), lambda i: (0, 0)),
    ],
    out_specs=pl.BlockSpec((ROWB, OUT), lambda i: (i, 0)),
    out_shape=jax.ShapeDtypeStruct((NP, OUT), jnp.float32),
)


def _fin_body(agg_ref, h_ref, dv_ref, b_ref, o_ref):
    o_ref[...] = ((agg_ref[0] + agg_ref[1] + h_ref[...]) * dv_ref[...]
                  + b_ref[...])


_tc_fin = pl.pallas_call(
    _fin_body,
    grid=(GRID,),
    in_specs=[
        pl.BlockSpec((NC, ROWB, OUT), lambda i: (0, i, 0)),
        pl.BlockSpec((ROWB, OUT), lambda i: (i, 0)),
        pl.BlockSpec((ROWB, 1), lambda i: (i, 0)),
        pl.BlockSpec((1, OUT), lambda i: (0, 0)),
    ],
    out_specs=pl.BlockSpec((ROWB, OUT), lambda i: (i, 0)),
    out_shape=jax.ShapeDtypeStruct((NP, OUT), jnp.float32),
)


def kernel(x, edge_index, W1, b1, W2, b2):
    ei = edge_index.astype(jnp.int32)
    pad = jnp.full((EP - E,), NP - 1, jnp.int32)
    src2d = jnp.concatenate([ei[0], pad]).reshape(EP // CHUNK, CHUNK)
    dst2d = jnp.concatenate([ei[1], pad]).reshape(EP // CHUNK, CHUNK)
    x_p = jnp.pad(x, ((0, NP - N), (0, 0)))

    p = _sc_deg(dst2d)
    dinv_col = _tc_dinv(p.reshape(2, NP // 128, 128)).reshape(NP, 1)
    h1p = _tc_mm1(x_p, W1, dinv_col)
    agg1 = _sc_agg_hid(src2d, dst2d, h1p)
    h2p = _tc_mid(agg1, h1p, dinv_col, b1.reshape(1, HID), W2)
    agg2 = _sc_agg_out(src2d, dst2d, h2p)
    out = _tc_fin(agg2, h2p, dinv_col, b2.reshape(1, OUT))
    return out[:N]
